# baseline (speedup 1.0000x reference)
import functools

import jax
import jax.numpy as jnp
from jax import lax
from jax.experimental import pallas as pl
from jax.experimental.pallas import tpu as pltpu
from jax.experimental.pallas import tpu_sc as plsc

N_NODES = 10000
HID = 128
ED = 16
NC, NS = 2, 16
NW = NC * NS
W = 128
NH = 2
NWIN = 40
EPT = W * NWIN
E_HALF = NW * EPT
E_PAD = NH * E_HALF
NACC = 10240
RPS = NACC // NS
BE = 1024
BN = 1000
f32 = jnp.float32

_mesh = plsc.VectorSubcoreMesh(core_axis_name="c", subcore_axis_name="s")
_sc_params = pltpu.CompilerParams(use_tc_tiling_on_sc=False)


def _silu(x):
    return x * jax.nn.sigmoid(x)


def _sc_gather2(a, b, idxr, idxc, d, tc_tiling):
    @functools.partial(
        pl.kernel,
        out_type=(
            jax.ShapeDtypeStruct((E_HALF, d), f32),
            jax.ShapeDtypeStruct((E_HALF, d), f32),
        ),
        mesh=_mesh,
        compiler_params=pltpu.CompilerParams(use_tc_tiling_on_sc=tc_tiling),
        scratch_types=[
            pltpu.VMEM((NWIN, W), jnp.int32),
            pltpu.VMEM((NWIN, W), jnp.int32),
            pltpu.VMEM((2, W, d), f32),
            pltpu.VMEM((2, W, d), f32),
            pltpu.SemaphoreType.DMA,
            pltpu.SemaphoreType.DMA,
        ],
    )
    def k(a_h, b_h, ir_h, ic_h, ga_h, gb_h,
          ir_v, ic_v, ra, rb, sem0, sem1):
        c = lax.axis_index("c")
        s = lax.axis_index("s")
        wid = s * NC + c
        pltpu.sync_copy(ir_h.at[wid], ir_v)
        pltpu.sync_copy(ic_h.at[wid], ic_v)
        base = wid * EPT
        sems = (sem0, sem1)

        def fire(w, sl):
            pltpu.async_copy(a_h.at[ir_v.at[w]], ra.at[sl], sems[sl])
            pltpu.async_copy(b_h.at[ic_v.at[w]], rb.at[sl], sems[sl])

        def drain(w, sl):
            pltpu.make_async_copy(a_h.at[ir_v.at[w]], ra.at[sl], sems[sl]).wait()
            pltpu.make_async_copy(b_h.at[ic_v.at[w]], rb.at[sl], sems[sl]).wait()

        def writeout(w, sl):
            off = base + w * W
            pltpu.sync_copy(ra.at[sl], ga_h.at[pl.ds(off, W)])
            pltpu.sync_copy(rb.at[sl], gb_h.at[pl.ds(off, W)])

        fire(0, 0)

        def body(i, carry):
            w1 = 2 * i + 1
            fire(w1, 1)
            drain(w1 - 1, 0)
            writeout(w1 - 1, 0)
            w2 = 2 * i + 2
            fire(w2, 0)
            drain(w2 - 1, 1)
            writeout(w2 - 1, 1)
            return carry

        if NWIN % 2:
            lax.fori_loop(0, (NWIN - 1) // 2, body, 0)
            drain(NWIN - 1, 0)
            writeout(NWIN - 1, 0)
        else:
            lax.fori_loop(0, (NWIN - 2) // 2, body, 0)
            fire(NWIN - 1, 1)
            drain(NWIN - 2, 0)
            writeout(NWIN - 2, 0)
            drain(NWIN - 1, 1)
            writeout(NWIN - 1, 1)

    return k(a, b, idxr, idxc)


def _sc_gather_pos(p, idxr, idxc):
    @functools.partial(
        pl.kernel,
        out_type=(
            jax.ShapeDtypeStruct((E_HALF // 8, 8 * 16), f32),
            jax.ShapeDtypeStruct((E_HALF // 8, 8 * 16), f32),
        ),
        mesh=_mesh,
        compiler_params=pltpu.CompilerParams(use_tc_tiling_on_sc=False),
        scratch_types=[
            pltpu.VMEM((NWIN, W), jnp.int32),
            pltpu.VMEM((NWIN, W), jnp.int32),
            pltpu.VMEM((2, W, 16), f32),
            pltpu.VMEM((2, W, 16), f32),
            pltpu.VMEM((W // 8, 8 * 16), f32),
            pltpu.VMEM((W // 8, 8 * 16), f32),
            pltpu.SemaphoreType.DMA,
            pltpu.SemaphoreType.DMA,
        ],
    )
    def k(p_h, ir_h, ic_h, pr_h, pc_h,
          ir_v, ic_v, ra, rb, ta, tb, sem0, sem1):
        c = lax.axis_index("c")
        s = lax.axis_index("s")
        wid = s * NC + c
        pltpu.sync_copy(ir_h.at[wid], ir_v)
        pltpu.sync_copy(ic_h.at[wid], ic_v)
        base = wid * EPT
        sems = (sem0, sem1)

        def fire(w, sl):
            pltpu.async_copy(p_h.at[ir_v.at[w]], ra.at[sl], sems[sl])
            pltpu.async_copy(p_h.at[ic_v.at[w]], rb.at[sl], sems[sl])

        def drain(w, sl):
            pltpu.make_async_copy(p_h.at[ir_v.at[w]], ra.at[sl], sems[sl]).wait()
            pltpu.make_async_copy(p_h.at[ic_v.at[w]], rb.at[sl], sems[sl]).wait()

        def writeout(w, sl):
            def pack(q, carry):
                for ci in range(8):
                    ta[q, pl.ds(16 * ci, 16)] = ra[sl, 8 * q + ci, :]
                    tb[q, pl.ds(16 * ci, 16)] = rb[sl, 8 * q + ci, :]
                return carry

            lax.fori_loop(0, W // 8, pack, 0)
            off8 = (base + w * W) // 8
            pltpu.sync_copy(ta, pr_h.at[pl.ds(off8, W // 8)])
            pltpu.sync_copy(tb, pc_h.at[pl.ds(off8, W // 8)])

        fire(0, 0)

        def body(i, carry):
            w1 = 2 * i + 1
            fire(w1, 1)
            drain(w1 - 1, 0)
            writeout(w1 - 1, 0)
            w2 = 2 * i + 2
            fire(w2, 0)
            drain(w2 - 1, 1)
            writeout(w2 - 1, 1)
            return carry

        if NWIN % 2:
            lax.fori_loop(0, (NWIN - 1) // 2, body, 0)
            drain(NWIN - 1, 0)
            writeout(NWIN - 1, 0)
        else:
            lax.fori_loop(0, (NWIN - 2) // 2, body, 0)
            fire(NWIN - 1, 1)
            drain(NWIN - 2, 0)
            writeout(NWIN - 2, 0)
            drain(NWIN - 1, 1)
            writeout(NWIN - 1, 1)

    return k(p, idxr, idxc)


def _sc_scatter(u, idxs, tc_tiling):
    d = u.shape[1]

    def body(u_h, ix_h, ou_h, ix_v, ub, acc_u, sem0, sem1):
        c = lax.axis_index("c")
        s = lax.axis_index("s")
        wid = s * NC + c
        r0 = s * RPS

        def zrow(i, carry):
            for j in range(d // 16):
                ub[0, i, pl.ds(j * 16, 16)] = jnp.zeros((16,), f32)
            return carry

        lax.fori_loop(0, W, zrow, 0)
        for k in range(RPS // W):
            pltpu.sync_copy(ub.at[0], acc_u.at[pl.ds(r0 + k * W, W)])
        pltpu.sync_copy(ix_h.at[wid], ix_v)
        plsc.subcore_barrier()
        sems = (sem0, sem1)

        def fire(w, sl):
            off = wid * EPT + w * W
            pltpu.async_copy(u_h.at[pl.ds(off, W)], ub.at[sl], sems[sl])

        def drain(w, sl):
            off = wid * EPT + w * W
            pltpu.make_async_copy(u_h.at[pl.ds(off, W)], ub.at[sl],
                                  sems[sl]).wait()

        def consume(w, sl):
            pltpu.sync_copy(ub.at[sl], acc_u.at[ix_v.at[w]], add=True)

        fire(0, 0)

        def w_body(i, carry):
            w1 = 2 * i + 1
            fire(w1, 1)
            drain(w1 - 1, 0)
            consume(w1 - 1, 0)
            w2 = 2 * i + 2
            fire(w2, 0)
            drain(w2 - 1, 1)
            consume(w2 - 1, 1)
            return carry

        if NWIN % 2:
            lax.fori_loop(0, (NWIN - 1) // 2, w_body, 0)
            drain(NWIN - 1, 0)
            consume(NWIN - 1, 0)
        else:
            lax.fori_loop(0, (NWIN - 2) // 2, w_body, 0)
            fire(NWIN - 1, 1)
            drain(NWIN - 2, 0)
            consume(NWIN - 2, 0)
            drain(NWIN - 1, 1)
            consume(NWIN - 1, 1)
        plsc.subcore_barrier()
        pltpu.sync_copy(acc_u.at[pl.ds(r0, RPS)], ou_h.at[c, pl.ds(r0, RPS)])

    kfn = functools.partial(
        pl.kernel,
        out_type=jax.ShapeDtypeStruct((NC, NACC, d), f32),
        mesh=_mesh,
        compiler_params=pltpu.CompilerParams(use_tc_tiling_on_sc=tc_tiling),
        scratch_types=[
            pltpu.VMEM((NWIN, W), jnp.int32),
            pltpu.VMEM((2, W, d), f32),
            pltpu.VMEM_SHARED((NACC, d), f32),
            pltpu.SemaphoreType.DMA,
            pltpu.SemaphoreType.DMA,
        ])(body)
    return kfn(u, idxs)


def _full(shape):
    return pl.BlockSpec(shape, lambda i: tuple(0 for _ in shape))


def _prep(x, wemb, bemb, w1a, w1b):
    def body(x_b, we, be, wa, wb, h_o, a_o, b_o):
        h = jnp.dot(x_b[...], we[...], preferred_element_type=f32) + be[...]
        h_o[...] = h
        a_o[...] = jnp.dot(h, wa[...], preferred_element_type=f32)
        b_o[...] = jnp.dot(h, wb[...], preferred_element_type=f32)

    n_spec = pl.BlockSpec((BN, HID), lambda i: (i, 0))
    return pl.pallas_call(
        body,
        grid=(N_NODES // BN,),
        in_specs=[n_spec, _full((HID, HID)), _full((1, HID)),
                  _full((HID, HID)), _full((HID, HID))],
        out_specs=[n_spec] * 3,
        out_shape=[jax.ShapeDtypeStruct((N_NODES, HID), f32)] * 3,
    )(x, wemb, bemb, w1a, w1b)


def _edge(ga, gb, prp, pcp, eap, masks, w1e_stack, b1, wr, w2, b2, coord):
    def body(*refs):
        if coord is not None:
            (ga_b, gb_b, prp_b, pcp_b, eap_b, sel_r, gm_r, pm_r, m8_r,
             w1es_, b1_, wr_, w2_, b2_, wc1_, bc1_, wc2_, u_o, t_o) = refs
        else:
            (ga_b, gb_b, prp_b, pcp_b, eap_b, sel_r, gm_r, pm_r, m8_r,
             w1es_, b1_, wr_, w2_, b2_, u_o) = refs
        sel = sel_r[...]
        gm = gm_r[...]

        dp = prp_b[...] - pcp_b[...]
        rdd = jnp.dot(sel, dp * dp, preferred_element_type=f32) * gm
        radial = jnp.sum(rdd, axis=1, keepdims=True)
        rea = jnp.dot(sel, eap_b[...], preferred_element_type=f32) * gm
        ea_term = jnp.dot(rea, w1es_[...], preferred_element_type=f32)
        pre = ga_b[...] + gb_b[...] + radial * wr_[...] + ea_term + b1_[...]
        u = _silu(pre)
        m = _silu(jnp.dot(u, w2_[...], preferred_element_type=f32) + b2_[...])
        u_o[...] = m
        if coord is not None:
            cc = _silu(jnp.dot(m, wc1_[...], preferred_element_type=f32)
                       + bc1_[...])
            sclr = jnp.sum(cc * wc2_[...], axis=1, keepdims=True)
            sg = lax.dot_general(sel, sclr * pm_r[...],
                                 (((0,), (0,)), ((), ())),
                                 preferred_element_type=f32)
            s16 = jnp.dot(sg, m8_r[...], preferred_element_type=f32)
            lane16 = lax.broadcasted_iota(jnp.int32, (BE // 8, HID), 1)
            t_o[...] = jnp.where(lane16 % 16 == 3, 1.0, dp * s16)

    e_spec = pl.BlockSpec((BE, HID), lambda i: (i, 0))
    p_spec = pl.BlockSpec((BE // 8, HID), lambda i: (i, 0))
    in_specs = [e_spec, e_spec, p_spec, p_spec, p_spec,
                _full((BE, BE // 8)), _full((BE, HID)), _full((BE, 8)),
                _full((8, HID)),
                _full((HID, HID)), _full((1, HID)), _full((1, HID)),
                _full((HID, HID)), _full((1, HID))]
    args = [ga, gb, prp, pcp, eap] + list(masks) + [w1e_stack, b1, wr, w2, b2]
    out_specs = [e_spec]
    out_shape = [jax.ShapeDtypeStruct((E_HALF, HID), f32)]
    if coord is not None:
        in_specs += [_full((HID, HID)), _full((1, HID)), _full((1, HID))]
        args += list(coord)
        out_specs.append(p_spec)
        out_shape.append(jax.ShapeDtypeStruct((E_HALF // 8, HID), f32))
    res = pl.pallas_call(
        body, grid=(E_HALF // BE,), in_specs=in_specs,
        out_specs=out_specs, out_shape=out_shape,
    )(*args)
    return res if coord is not None else res[0]


def _node1(pu, pt, h, posp, wna, wnb, bn1, wn2, bn2, wa2, wb2):
    def body(pu_b, pt_b, h_b, pp_b, wna_, wnb_, bn1_, wn2_, bn2_,
             wa2_, wb2_, h2_o, a2_o, b2_o, p2_o):
        agg = pu_b[0] + pu_b[1] + pu_b[2] + pu_b[3]
        t = pt_b[0] + pt_b[1] + pt_b[2] + pt_b[3]
        cnt = jnp.maximum(t[:, 3:4], 1.0)
        lane = lax.broadcasted_iota(jnp.int32, (BN, 16), 1)
        p2_o[...] = pp_b[...] + jnp.where(lane < 3, t / cnt, 0.0)
        pre = (jnp.dot(h_b[...], wna_[...], preferred_element_type=f32)
               + jnp.dot(agg, wnb_[...], preferred_element_type=f32)
               + bn1_[...])
        hn = (jnp.dot(_silu(pre), wn2_[...], preferred_element_type=f32)
              + bn2_[...])
        h2 = h_b[...] + hn
        h2_o[...] = h2
        a2_o[...] = jnp.dot(h2, wa2_[...], preferred_element_type=f32)
        b2_o[...] = jnp.dot(h2, wb2_[...], preferred_element_type=f32)

    n_spec = pl.BlockSpec((BN, HID), lambda i: (i, 0))
    s_spec = pl.BlockSpec((BN, 16), lambda i: (i, 0))
    pu_spec = pl.BlockSpec((NH * NC, BN, HID), lambda i: (0, i, 0))
    pt_spec = pl.BlockSpec((NH * NC, BN, 16), lambda i: (0, i, 0))
    return pl.pallas_call(
        body,
        grid=(N_NODES // BN,),
        in_specs=[pu_spec, pt_spec, n_spec, s_spec,
                  _full((HID, HID)), _full((HID, HID)), _full((1, HID)),
                  _full((HID, HID)), _full((1, HID)),
                  _full((HID, HID)), _full((HID, HID))],
        out_specs=[n_spec, n_spec, n_spec, s_spec],
        out_shape=[jax.ShapeDtypeStruct((N_NODES, HID), f32)] * 3
        + [jax.ShapeDtypeStruct((N_NODES, 16), f32)],
    )(pu, pt, h, posp, wna, wnb, bn1, wn2, bn2, wa2, wb2)


def _node2(pu, h, wna, wnb, bn1, wn2, bn2, wo, bo):
    def body(pu_b, h_b, wna_, wnb_, bn1_, wn2_, bn2_, wo_, bo_, out_o):
        agg = pu_b[0] + pu_b[1] + pu_b[2] + pu_b[3]
        pre = (jnp.dot(h_b[...], wna_[...], preferred_element_type=f32)
               + jnp.dot(agg, wnb_[...], preferred_element_type=f32)
               + bn1_[...])
        hn = (jnp.dot(_silu(pre), wn2_[...], preferred_element_type=f32)
              + bn2_[...])
        h2 = h_b[...] + hn
        out_o[...] = jnp.dot(h2, wo_[...], preferred_element_type=f32) + bo_[...]

    n_spec = pl.BlockSpec((BN, HID), lambda i: (i, 0))
    pu_spec = pl.BlockSpec((NH * NC, BN, HID), lambda i: (0, i, 0))
    return pl.pallas_call(
        body,
        grid=(N_NODES // BN,),
        in_specs=[pu_spec, n_spec,
                  _full((HID, HID)), _full((HID, HID)), _full((1, HID)),
                  _full((HID, HID)), _full((1, HID)),
                  _full((HID, HID)), _full((1, HID))],
        out_specs=[n_spec],
        out_shape=[jax.ShapeDtypeStruct((N_NODES, HID), f32)],
    )(pu, h, wna, wnb, bn1, wn2, bn2, wo, bo)[0]


def _row(v):
    return v.reshape(1, -1)


def kernel(x, pos, edge_attr, params, edge_index):
    row, col = edge_index[0], edge_index[1]
    e = row.shape[0]
    npad = E_PAD - e
    padg = (jnp.arange(npad, dtype=jnp.int32) * 97) % N_NODES
    pads = N_NODES + jnp.arange(npad, dtype=jnp.int32) % (NACC - N_NODES)
    idx_shape = (NH, NW, NWIN, W)
    rowg = jnp.concatenate([row, padg]).reshape(idx_shape)
    colg = jnp.concatenate([col, padg]).reshape(idx_shape)
    rows = jnp.concatenate([row, pads]).reshape(idx_shape)
    eap = jnp.concatenate(
        [jnp.reshape(edge_attr, (e // 8, 8 * ED)),
         jnp.zeros((npad // 8, 8 * ED), f32)],
        axis=0).reshape(NH, E_HALF // 8, 8 * ED)
    posp = jnp.concatenate([pos, jnp.zeros((N_NODES, 13), f32)], axis=1)

    e_i = lax.broadcasted_iota(jnp.int32, (BE, BE // 8), 0)
    r_i = lax.broadcasted_iota(jnp.int32, (BE, BE // 8), 1)
    sel = jnp.where(e_i // 8 == r_i, 1.0, 0.0).astype(f32)
    e_j = lax.broadcasted_iota(jnp.int32, (BE, HID), 0)
    c_j = lax.broadcasted_iota(jnp.int32, (BE, HID), 1)
    gm = jnp.where(c_j // 16 == e_j % 8, 1.0, 0.0).astype(f32)
    g_i = lax.broadcasted_iota(jnp.int32, (BE, 8), 1)
    e_k = lax.broadcasted_iota(jnp.int32, (BE, 8), 0)
    pm = jnp.where(e_k % 8 == g_i, 1.0, 0.0).astype(f32)
    c_m = lax.broadcasted_iota(jnp.int32, (8, HID), 1)
    g_m = lax.broadcasted_iota(jnp.int32, (8, HID), 0)
    m8 = jnp.where(c_m // 16 == g_m, 1.0, 0.0).astype(f32)
    masks = (sel, gm, pm, m8)

    lw = []
    for lp in params["layers"]:
        w1 = lp["edge_mlp"][0]["W"]
        lw.append(dict(
            wa=w1[:HID], wb=w1[HID:2 * HID], wr=_row(w1[2 * HID]),
            w1es=jnp.concatenate([w1[2 * HID + 1:]] * 8, axis=0),
            b1=_row(lp["edge_mlp"][0]["b"]),
            w2=lp["edge_mlp"][1]["W"], b2=_row(lp["edge_mlp"][1]["b"]),
            wc1=lp["coord_mlp"][0]["W"], bc1=_row(lp["coord_mlp"][0]["b"]),
            wc2=_row(lp["coord_mlp"][1]["W"][:, 0]),
            wna=lp["node_mlp"][0]["W"][:HID],
            wnb=lp["node_mlp"][0]["W"][HID:],
            bn1=_row(lp["node_mlp"][0]["b"]),
            wn2=lp["node_mlp"][1]["W"], bn2=_row(lp["node_mlp"][1]["b"]),
        ))

    h, a1, b1t = _prep(x, params["emb_in"]["W"], _row(params["emb_in"]["b"]),
                       lw[0]["wa"], lw[0]["wb"])

    pu1, pt1 = [], []
    for hh in range(NH):
        ga, gb = _sc_gather2(a1, b1t, rowg[hh], colg[hh], HID, True)
        prg, pcg = _sc_gather_pos(posp, rowg[hh], colg[hh])
        u1, t1 = _edge(ga, gb, prg, pcg,
                       eap[hh], masks, lw[0]["w1es"],
                       lw[0]["b1"], lw[0]["wr"], lw[0]["w2"], lw[0]["b2"],
                       (lw[0]["wc1"], lw[0]["bc1"], lw[0]["wc2"]))
        pu1.append(_sc_scatter(u1, rows[hh], True))
        pt1.append(_sc_scatter(jnp.reshape(t1, (E_HALF, 16)), rows[hh],
                               False))
    h2, a2, b2t, posp2 = _node1(jnp.concatenate(pu1), jnp.concatenate(pt1),
                                h, posp,
                                lw[0]["wna"], lw[0]["wnb"], lw[0]["bn1"],
                                lw[0]["wn2"], lw[0]["bn2"],
                                lw[1]["wa"], lw[1]["wb"])

    pu2 = []
    for hh in range(NH):
        ga2, gb2 = _sc_gather2(a2, b2t, rowg[hh], colg[hh], HID, True)
        prg2, pcg2 = _sc_gather_pos(posp2, rowg[hh], colg[hh])
        u2 = _edge(ga2, gb2, prg2, pcg2,
                   eap[hh], masks, lw[1]["w1es"], lw[1]["b1"],
                   lw[1]["wr"], lw[1]["w2"], lw[1]["b2"], None)
        pu2.append(_sc_scatter(u2, rows[hh], True))
    out = _node2(jnp.concatenate(pu2), h2,
                 lw[1]["wna"], lw[1]["wnb"], lw[1]["bn1"],
                 lw[1]["wn2"], lw[1]["bn2"],
                 params["emb_out"]["W"], _row(params["emb_out"]["b"]))
    return out


# speedup vs baseline: 1.0056x; 1.0056x over previous
import functools

import jax
import jax.numpy as jnp
from jax import lax
from jax.experimental import pallas as pl
from jax.experimental.pallas import tpu as pltpu
from jax.experimental.pallas import tpu_sc as plsc

N_NODES = 10000
HID = 128
ED = 16
NC, NS = 2, 16
NW = NC * NS
W = 128
NH = 2
NWIN = 40
EPT = W * NWIN
E_HALF = NW * EPT
E_PAD = NH * E_HALF
NACC = 10240
RPS = NACC // NS
BE = 1024
BN = 1000
f32 = jnp.float32

_mesh = plsc.VectorSubcoreMesh(core_axis_name="c", subcore_axis_name="s")
_sc_params = pltpu.CompilerParams(use_tc_tiling_on_sc=False)


def _silu(x):
    return x * jax.nn.sigmoid(x)


def _sc_gather2(a, b, idxr, idxc, d, tc_tiling):
    @functools.partial(
        pl.kernel,
        out_type=(
            jax.ShapeDtypeStruct((E_HALF, d), f32),
            jax.ShapeDtypeStruct((E_HALF, d), f32),
        ),
        mesh=_mesh,
        compiler_params=pltpu.CompilerParams(use_tc_tiling_on_sc=tc_tiling),
        scratch_types=[
            pltpu.VMEM((NWIN, W), jnp.int32),
            pltpu.VMEM((NWIN, W), jnp.int32),
            pltpu.VMEM((2, W, d), f32),
            pltpu.VMEM((2, W, d), f32),
            pltpu.SemaphoreType.DMA,
            pltpu.SemaphoreType.DMA,
        ],
    )
    def k(a_h, b_h, ir_h, ic_h, ga_h, gb_h,
          ir_v, ic_v, ra, rb, sem0, sem1):
        c = lax.axis_index("c")
        s = lax.axis_index("s")
        wid = s * NC + c
        pltpu.sync_copy(ir_h.at[wid], ir_v)
        pltpu.sync_copy(ic_h.at[wid], ic_v)
        base = wid * EPT
        sems = (sem0, sem1)

        def fire(w, sl):
            pltpu.async_copy(a_h.at[ir_v.at[w]], ra.at[sl], sems[sl])
            pltpu.async_copy(b_h.at[ic_v.at[w]], rb.at[sl], sems[sl])

        def drain(w, sl):
            pltpu.make_async_copy(a_h.at[ir_v.at[w]], ra.at[sl], sems[sl]).wait()
            pltpu.make_async_copy(b_h.at[ic_v.at[w]], rb.at[sl], sems[sl]).wait()

        def writeout(w, sl):
            off = base + w * W
            pltpu.sync_copy(ra.at[sl], ga_h.at[pl.ds(off, W)])
            pltpu.sync_copy(rb.at[sl], gb_h.at[pl.ds(off, W)])

        fire(0, 0)

        def body(i, carry):
            w1 = 2 * i + 1
            fire(w1, 1)
            drain(w1 - 1, 0)
            writeout(w1 - 1, 0)
            w2 = 2 * i + 2
            fire(w2, 0)
            drain(w2 - 1, 1)
            writeout(w2 - 1, 1)
            return carry

        if NWIN % 2:
            lax.fori_loop(0, (NWIN - 1) // 2, body, 0)
            drain(NWIN - 1, 0)
            writeout(NWIN - 1, 0)
        else:
            lax.fori_loop(0, (NWIN - 2) // 2, body, 0)
            fire(NWIN - 1, 1)
            drain(NWIN - 2, 0)
            writeout(NWIN - 2, 0)
            drain(NWIN - 1, 1)
            writeout(NWIN - 1, 1)

    return k(a, b, idxr, idxc)


def _sc_gather_pos(p, idxr, idxc):
    @functools.partial(
        pl.kernel,
        out_type=(
            jax.ShapeDtypeStruct((E_HALF // 8, 8 * 16), f32),
            jax.ShapeDtypeStruct((E_HALF // 8, 8 * 16), f32),
        ),
        mesh=_mesh,
        compiler_params=pltpu.CompilerParams(use_tc_tiling_on_sc=False),
        scratch_types=[
            pltpu.VMEM((NWIN, W), jnp.int32),
            pltpu.VMEM((NWIN, W), jnp.int32),
            pltpu.VMEM((2, W, 16), f32),
            pltpu.VMEM((2, W, 16), f32),
            pltpu.VMEM((W // 8, 8 * 16), f32),
            pltpu.VMEM((W // 8, 8 * 16), f32),
            pltpu.SemaphoreType.DMA,
            pltpu.SemaphoreType.DMA,
        ],
    )
    def k(p_h, ir_h, ic_h, pr_h, pc_h,
          ir_v, ic_v, ra, rb, ta, tb, sem0, sem1):
        c = lax.axis_index("c")
        s = lax.axis_index("s")
        wid = s * NC + c
        pltpu.sync_copy(ir_h.at[wid], ir_v)
        pltpu.sync_copy(ic_h.at[wid], ic_v)
        base = wid * EPT
        sems = (sem0, sem1)

        def fire(w, sl):
            pltpu.async_copy(p_h.at[ir_v.at[w]], ra.at[sl], sems[sl])
            pltpu.async_copy(p_h.at[ic_v.at[w]], rb.at[sl], sems[sl])

        def drain(w, sl):
            pltpu.make_async_copy(p_h.at[ir_v.at[w]], ra.at[sl], sems[sl]).wait()
            pltpu.make_async_copy(p_h.at[ic_v.at[w]], rb.at[sl], sems[sl]).wait()

        def writeout(w, sl):
            def pack(q, carry):
                for ci in range(8):
                    ta[q, pl.ds(16 * ci, 16)] = ra[sl, 8 * q + ci, :]
                    tb[q, pl.ds(16 * ci, 16)] = rb[sl, 8 * q + ci, :]
                return carry

            lax.fori_loop(0, W // 8, pack, 0)
            off8 = (base + w * W) // 8
            pltpu.sync_copy(ta, pr_h.at[pl.ds(off8, W // 8)])
            pltpu.sync_copy(tb, pc_h.at[pl.ds(off8, W // 8)])

        fire(0, 0)

        def body(i, carry):
            w1 = 2 * i + 1
            fire(w1, 1)
            drain(w1 - 1, 0)
            writeout(w1 - 1, 0)
            w2 = 2 * i + 2
            fire(w2, 0)
            drain(w2 - 1, 1)
            writeout(w2 - 1, 1)
            return carry

        if NWIN % 2:
            lax.fori_loop(0, (NWIN - 1) // 2, body, 0)
            drain(NWIN - 1, 0)
            writeout(NWIN - 1, 0)
        else:
            lax.fori_loop(0, (NWIN - 2) // 2, body, 0)
            fire(NWIN - 1, 1)
            drain(NWIN - 2, 0)
            writeout(NWIN - 2, 0)
            drain(NWIN - 1, 1)
            writeout(NWIN - 1, 1)

    return k(p, idxr, idxc)


def _sc_scatter(u, idxs, tc_tiling):
    d = u.shape[1]

    def body(u_h, ix_h, ou_h, ix_v, ub, acc_u, sem0, sem1):
        c = lax.axis_index("c")
        s = lax.axis_index("s")
        wid = s * NC + c
        r0 = s * RPS

        def zrow(i, carry):
            for j in range(d // 16):
                ub[0, i, pl.ds(j * 16, 16)] = jnp.zeros((16,), f32)
            return carry

        lax.fori_loop(0, W, zrow, 0)
        for k in range(RPS // W):
            pltpu.sync_copy(ub.at[0], acc_u.at[pl.ds(r0 + k * W, W)])
        pltpu.sync_copy(ix_h.at[wid], ix_v)
        plsc.subcore_barrier()
        sems = (sem0, sem1)

        def fire(w, sl):
            off = wid * EPT + w * W
            pltpu.async_copy(u_h.at[pl.ds(off, W)], ub.at[sl], sems[sl])

        def drain(w, sl):
            off = wid * EPT + w * W
            pltpu.make_async_copy(u_h.at[pl.ds(off, W)], ub.at[sl],
                                  sems[sl]).wait()

        def consume(w, sl):
            pltpu.sync_copy(ub.at[sl], acc_u.at[ix_v.at[w]], add=True)

        fire(0, 0)

        def w_body(i, carry):
            w1 = 2 * i + 1
            fire(w1, 1)
            drain(w1 - 1, 0)
            consume(w1 - 1, 0)
            w2 = 2 * i + 2
            fire(w2, 0)
            drain(w2 - 1, 1)
            consume(w2 - 1, 1)
            return carry

        if NWIN % 2:
            lax.fori_loop(0, (NWIN - 1) // 2, w_body, 0)
            drain(NWIN - 1, 0)
            consume(NWIN - 1, 0)
        else:
            lax.fori_loop(0, (NWIN - 2) // 2, w_body, 0)
            fire(NWIN - 1, 1)
            drain(NWIN - 2, 0)
            consume(NWIN - 2, 0)
            drain(NWIN - 1, 1)
            consume(NWIN - 1, 1)
        plsc.subcore_barrier()
        pltpu.sync_copy(acc_u.at[pl.ds(r0, RPS)], ou_h.at[c, pl.ds(r0, RPS)])

    kfn = functools.partial(
        pl.kernel,
        out_type=jax.ShapeDtypeStruct((NC, NACC, d), f32),
        mesh=_mesh,
        compiler_params=pltpu.CompilerParams(use_tc_tiling_on_sc=tc_tiling),
        scratch_types=[
            pltpu.VMEM((NWIN, W), jnp.int32),
            pltpu.VMEM((2, W, d), f32),
            pltpu.VMEM_SHARED((NACC, d), f32),
            pltpu.SemaphoreType.DMA,
            pltpu.SemaphoreType.DMA,
        ])(body)
    return kfn(u, idxs)


def _full(shape):
    return pl.BlockSpec(shape, lambda i: tuple(0 for _ in shape))


def _prep(x, wemb, bemb, w1a, w1b):
    def body(x_b, we, be, wa, wb, h_o, a_o, b_o):
        h = jnp.dot(x_b[...], we[...], preferred_element_type=f32) + be[...]
        h_o[...] = h
        a_o[...] = jnp.dot(h, wa[...], preferred_element_type=f32)
        b_o[...] = jnp.dot(h, wb[...], preferred_element_type=f32)

    n_spec = pl.BlockSpec((BN, HID), lambda i: (i, 0))
    return pl.pallas_call(
        body,
        grid=(N_NODES // BN,),
        in_specs=[n_spec, _full((HID, HID)), _full((1, HID)),
                  _full((HID, HID)), _full((HID, HID))],
        out_specs=[n_spec] * 3,
        out_shape=[jax.ShapeDtypeStruct((N_NODES, HID), f32)] * 3,
    )(x, wemb, bemb, w1a, w1b)


def _edge(ga, gb, prp, pcp, eap, masks, w1e_stack, b1, wr, w2, b2, coord):
    def body(*refs):
        if coord is not None:
            (ga_b, gb_b, prp_b, pcp_b, eap_b, sel_r, gm_r, pm_r, m8_r,
             w1es_, b1_, wr_, w2_, b2_, wc1_, bc1_, wc2_, u_o, t_o) = refs
        else:
            (ga_b, gb_b, prp_b, pcp_b, eap_b, sel_r, gm_r, pm_r, m8_r,
             w1es_, b1_, wr_, w2_, b2_, u_o) = refs
        bf16 = jnp.bfloat16
        sel = sel_r[...].astype(bf16)
        gm = gm_r[...]

        dp = prp_b[...] - pcp_b[...]
        rdd = jnp.dot(sel, (dp * dp).astype(bf16),
                      preferred_element_type=f32) * gm
        radial = jnp.sum(rdd, axis=1, keepdims=True)
        rea = jnp.dot(sel, eap_b[...].astype(bf16),
                      preferred_element_type=f32) * gm
        ea_term = jnp.dot(rea.astype(bf16), w1es_[...].astype(bf16),
                          preferred_element_type=f32)
        pre = ga_b[...] + gb_b[...] + radial * wr_[...] + ea_term + b1_[...]
        u = _silu(pre)
        m = _silu(jnp.dot(u.astype(bf16), w2_[...].astype(bf16),
                          preferred_element_type=f32) + b2_[...])
        u_o[...] = m
        if coord is not None:
            cc = _silu(jnp.dot(m.astype(bf16), wc1_[...].astype(bf16),
                               preferred_element_type=f32) + bc1_[...])
            sclr = jnp.sum(cc * wc2_[...], axis=1, keepdims=True)
            sg = lax.dot_general(sel, sclr * pm_r[...],
                                 (((0,), (0,)), ((), ())),
                                 preferred_element_type=f32)
            s16 = jnp.dot(sg, m8_r[...], preferred_element_type=f32)
            lane16 = lax.broadcasted_iota(jnp.int32, (BE // 8, HID), 1)
            t_o[...] = jnp.where(lane16 % 16 == 3, 1.0, dp * s16)

    e_spec = pl.BlockSpec((BE, HID), lambda i: (i, 0))
    p_spec = pl.BlockSpec((BE // 8, HID), lambda i: (i, 0))
    in_specs = [e_spec, e_spec, p_spec, p_spec, p_spec,
                _full((BE, BE // 8)), _full((BE, HID)), _full((BE, 8)),
                _full((8, HID)),
                _full((HID, HID)), _full((1, HID)), _full((1, HID)),
                _full((HID, HID)), _full((1, HID))]
    args = [ga, gb, prp, pcp, eap] + list(masks) + [w1e_stack, b1, wr, w2, b2]
    out_specs = [e_spec]
    out_shape = [jax.ShapeDtypeStruct((E_HALF, HID), f32)]
    if coord is not None:
        in_specs += [_full((HID, HID)), _full((1, HID)), _full((1, HID))]
        args += list(coord)
        out_specs.append(p_spec)
        out_shape.append(jax.ShapeDtypeStruct((E_HALF // 8, HID), f32))
    res = pl.pallas_call(
        body, grid=(E_HALF // BE,), in_specs=in_specs,
        out_specs=out_specs, out_shape=out_shape,
    )(*args)
    return res if coord is not None else res[0]


def _node1(pu, pt, h, posp, wna, wnb, bn1, wn2, bn2, wa2, wb2):
    def body(pu_b, pt_b, h_b, pp_b, wna_, wnb_, bn1_, wn2_, bn2_,
             wa2_, wb2_, h2_o, a2_o, b2_o, p2_o):
        agg = pu_b[0] + pu_b[1] + pu_b[2] + pu_b[3]
        t = pt_b[0] + pt_b[1] + pt_b[2] + pt_b[3]
        cnt = jnp.maximum(t[:, 3:4], 1.0)
        lane = lax.broadcasted_iota(jnp.int32, (BN, 16), 1)
        p2_o[...] = pp_b[...] + jnp.where(lane < 3, t / cnt, 0.0)
        pre = (jnp.dot(h_b[...], wna_[...], preferred_element_type=f32)
               + jnp.dot(agg, wnb_[...], preferred_element_type=f32)
               + bn1_[...])
        hn = (jnp.dot(_silu(pre), wn2_[...], preferred_element_type=f32)
              + bn2_[...])
        h2 = h_b[...] + hn
        h2_o[...] = h2
        a2_o[...] = jnp.dot(h2, wa2_[...], preferred_element_type=f32)
        b2_o[...] = jnp.dot(h2, wb2_[...], preferred_element_type=f32)

    n_spec = pl.BlockSpec((BN, HID), lambda i: (i, 0))
    s_spec = pl.BlockSpec((BN, 16), lambda i: (i, 0))
    pu_spec = pl.BlockSpec((NH * NC, BN, HID), lambda i: (0, i, 0))
    pt_spec = pl.BlockSpec((NH * NC, BN, 16), lambda i: (0, i, 0))
    return pl.pallas_call(
        body,
        grid=(N_NODES // BN,),
        in_specs=[pu_spec, pt_spec, n_spec, s_spec,
                  _full((HID, HID)), _full((HID, HID)), _full((1, HID)),
                  _full((HID, HID)), _full((1, HID)),
                  _full((HID, HID)), _full((HID, HID))],
        out_specs=[n_spec, n_spec, n_spec, s_spec],
        out_shape=[jax.ShapeDtypeStruct((N_NODES, HID), f32)] * 3
        + [jax.ShapeDtypeStruct((N_NODES, 16), f32)],
    )(pu, pt, h, posp, wna, wnb, bn1, wn2, bn2, wa2, wb2)


def _node2(pu, h, wna, wnb, bn1, wn2, bn2, wo, bo):
    def body(pu_b, h_b, wna_, wnb_, bn1_, wn2_, bn2_, wo_, bo_, out_o):
        agg = pu_b[0] + pu_b[1] + pu_b[2] + pu_b[3]
        pre = (jnp.dot(h_b[...], wna_[...], preferred_element_type=f32)
               + jnp.dot(agg, wnb_[...], preferred_element_type=f32)
               + bn1_[...])
        hn = (jnp.dot(_silu(pre), wn2_[...], preferred_element_type=f32)
              + bn2_[...])
        h2 = h_b[...] + hn
        out_o[...] = jnp.dot(h2, wo_[...], preferred_element_type=f32) + bo_[...]

    n_spec = pl.BlockSpec((BN, HID), lambda i: (i, 0))
    pu_spec = pl.BlockSpec((NH * NC, BN, HID), lambda i: (0, i, 0))
    return pl.pallas_call(
        body,
        grid=(N_NODES // BN,),
        in_specs=[pu_spec, n_spec,
                  _full((HID, HID)), _full((HID, HID)), _full((1, HID)),
                  _full((HID, HID)), _full((1, HID)),
                  _full((HID, HID)), _full((1, HID))],
        out_specs=[n_spec],
        out_shape=[jax.ShapeDtypeStruct((N_NODES, HID), f32)],
    )(pu, h, wna, wnb, bn1, wn2, bn2, wo, bo)[0]


def _row(v):
    return v.reshape(1, -1)


def kernel(x, pos, edge_attr, params, edge_index):
    row, col = edge_index[0], edge_index[1]
    e = row.shape[0]
    npad = E_PAD - e
    padg = (jnp.arange(npad, dtype=jnp.int32) * 97) % N_NODES
    pads = N_NODES + jnp.arange(npad, dtype=jnp.int32) % (NACC - N_NODES)
    idx_shape = (NH, NW, NWIN, W)
    rowg = jnp.concatenate([row, padg]).reshape(idx_shape)
    colg = jnp.concatenate([col, padg]).reshape(idx_shape)
    rows = jnp.concatenate([row, pads]).reshape(idx_shape)
    eap = jnp.concatenate(
        [jnp.reshape(edge_attr, (e // 8, 8 * ED)),
         jnp.zeros((npad // 8, 8 * ED), f32)],
        axis=0).reshape(NH, E_HALF // 8, 8 * ED)
    posp = jnp.concatenate([pos, jnp.zeros((N_NODES, 13), f32)], axis=1)

    e_i = lax.broadcasted_iota(jnp.int32, (BE, BE // 8), 0)
    r_i = lax.broadcasted_iota(jnp.int32, (BE, BE // 8), 1)
    sel = jnp.where(e_i // 8 == r_i, 1.0, 0.0).astype(f32)
    e_j = lax.broadcasted_iota(jnp.int32, (BE, HID), 0)
    c_j = lax.broadcasted_iota(jnp.int32, (BE, HID), 1)
    gm = jnp.where(c_j // 16 == e_j % 8, 1.0, 0.0).astype(f32)
    g_i = lax.broadcasted_iota(jnp.int32, (BE, 8), 1)
    e_k = lax.broadcasted_iota(jnp.int32, (BE, 8), 0)
    pm = jnp.where(e_k % 8 == g_i, 1.0, 0.0).astype(f32)
    c_m = lax.broadcasted_iota(jnp.int32, (8, HID), 1)
    g_m = lax.broadcasted_iota(jnp.int32, (8, HID), 0)
    m8 = jnp.where(c_m // 16 == g_m, 1.0, 0.0).astype(f32)
    masks = (sel, gm, pm, m8)

    lw = []
    for lp in params["layers"]:
        w1 = lp["edge_mlp"][0]["W"]
        lw.append(dict(
            wa=w1[:HID], wb=w1[HID:2 * HID], wr=_row(w1[2 * HID]),
            w1es=jnp.concatenate([w1[2 * HID + 1:]] * 8, axis=0),
            b1=_row(lp["edge_mlp"][0]["b"]),
            w2=lp["edge_mlp"][1]["W"], b2=_row(lp["edge_mlp"][1]["b"]),
            wc1=lp["coord_mlp"][0]["W"], bc1=_row(lp["coord_mlp"][0]["b"]),
            wc2=_row(lp["coord_mlp"][1]["W"][:, 0]),
            wna=lp["node_mlp"][0]["W"][:HID],
            wnb=lp["node_mlp"][0]["W"][HID:],
            bn1=_row(lp["node_mlp"][0]["b"]),
            wn2=lp["node_mlp"][1]["W"], bn2=_row(lp["node_mlp"][1]["b"]),
        ))

    h, a1, b1t = _prep(x, params["emb_in"]["W"], _row(params["emb_in"]["b"]),
                       lw[0]["wa"], lw[0]["wb"])

    pu1, pt1 = [], []
    for hh in range(NH):
        ga, gb = _sc_gather2(a1, b1t, rowg[hh], colg[hh], HID, True)
        prg, pcg = _sc_gather_pos(posp, rowg[hh], colg[hh])
        u1, t1 = _edge(ga, gb, prg, pcg,
                       eap[hh], masks, lw[0]["w1es"],
                       lw[0]["b1"], lw[0]["wr"], lw[0]["w2"], lw[0]["b2"],
                       (lw[0]["wc1"], lw[0]["bc1"], lw[0]["wc2"]))
        pu1.append(_sc_scatter(u1, rows[hh], True))
        pt1.append(_sc_scatter(jnp.reshape(t1, (E_HALF, 16)), rows[hh],
                               False))
    h2, a2, b2t, posp2 = _node1(jnp.concatenate(pu1), jnp.concatenate(pt1),
                                h, posp,
                                lw[0]["wna"], lw[0]["wnb"], lw[0]["bn1"],
                                lw[0]["wn2"], lw[0]["bn2"],
                                lw[1]["wa"], lw[1]["wb"])

    pu2 = []
    for hh in range(NH):
        ga2, gb2 = _sc_gather2(a2, b2t, rowg[hh], colg[hh], HID, True)
        prg2, pcg2 = _sc_gather_pos(posp2, rowg[hh], colg[hh])
        u2 = _edge(ga2, gb2, prg2, pcg2,
                   eap[hh], masks, lw[1]["w1es"], lw[1]["b1"],
                   lw[1]["wr"], lw[1]["w2"], lw[1]["b2"], None)
        pu2.append(_sc_scatter(u2, rows[hh], True))
    out = _node2(jnp.concatenate(pu2), h2,
                 lw[1]["wna"], lw[1]["wnb"], lw[1]["bn1"],
                 lw[1]["wn2"], lw[1]["bn2"],
                 params["emb_out"]["W"], _row(params["emb_out"]["b"]))
    return out


# speedup vs baseline: 1.1172x; 1.1109x over previous
import functools

import jax
import jax.numpy as jnp
from jax import lax
from jax.experimental import pallas as pl
from jax.experimental.pallas import tpu as pltpu
from jax.experimental.pallas import tpu_sc as plsc

N_NODES = 10000
HID = 128
ED = 16
NC, NS = 2, 16
NW = NC * NS
W = 128
NH = 2
NWIN = 40
EPT = W * NWIN
E_HALF = NW * EPT
E_PAD = NH * E_HALF
NACC = 10240
RPS = NACC // NS
BE = 1024
BN = 1000
f32 = jnp.float32

_mesh = plsc.VectorSubcoreMesh(core_axis_name="c", subcore_axis_name="s")
_sc_params = pltpu.CompilerParams(use_tc_tiling_on_sc=False)


def _silu(x):
    return x * jax.nn.sigmoid(x)


def _sc_gather2(a, b, idxr, idxc, d, tc_tiling):
    @functools.partial(
        pl.kernel,
        out_type=jax.ShapeDtypeStruct((E_HALF, d), f32),
        mesh=_mesh,
        compiler_params=pltpu.CompilerParams(use_tc_tiling_on_sc=tc_tiling),
        scratch_types=[
            pltpu.VMEM((NWIN, W), jnp.int32),
            pltpu.VMEM((NWIN, W), jnp.int32),
            pltpu.VMEM((2, W, d), f32),
            pltpu.VMEM((2, W, d), f32),
            pltpu.SemaphoreType.DMA,
            pltpu.SemaphoreType.DMA,
        ],
    )
    def k(a_h, b_h, ir_h, ic_h, ga_h,
          ir_v, ic_v, ra, rb, sem0, sem1):
        c = lax.axis_index("c")
        s = lax.axis_index("s")
        wid = s * NC + c
        pltpu.sync_copy(ir_h.at[wid], ir_v)
        pltpu.sync_copy(ic_h.at[wid], ic_v)
        base = wid * EPT
        sems = (sem0, sem1)

        def fire(w, sl):
            pltpu.async_copy(a_h.at[ir_v.at[w]], ra.at[sl], sems[sl])
            pltpu.async_copy(b_h.at[ic_v.at[w]], rb.at[sl], sems[sl])

        def drain(w, sl):
            pltpu.make_async_copy(a_h.at[ir_v.at[w]], ra.at[sl], sems[sl]).wait()
            pltpu.make_async_copy(b_h.at[ic_v.at[w]], rb.at[sl], sems[sl]).wait()

        def writeout(w, sl):
            def addrow(i, carry):
                for j in range(d // 16):
                    ra[sl, i, pl.ds(16 * j, 16)] = (
                        ra[sl, i, pl.ds(16 * j, 16)]
                        + rb[sl, i, pl.ds(16 * j, 16)])
                return carry

            lax.fori_loop(0, W, addrow, 0)
            off = base + w * W
            pltpu.sync_copy(ra.at[sl], ga_h.at[pl.ds(off, W)])

        fire(0, 0)

        def body(i, carry):
            w1 = 2 * i + 1
            fire(w1, 1)
            drain(w1 - 1, 0)
            writeout(w1 - 1, 0)
            w2 = 2 * i + 2
            fire(w2, 0)
            drain(w2 - 1, 1)
            writeout(w2 - 1, 1)
            return carry

        if NWIN % 2:
            lax.fori_loop(0, (NWIN - 1) // 2, body, 0)
            drain(NWIN - 1, 0)
            writeout(NWIN - 1, 0)
        else:
            lax.fori_loop(0, (NWIN - 2) // 2, body, 0)
            fire(NWIN - 1, 1)
            drain(NWIN - 2, 0)
            writeout(NWIN - 2, 0)
            drain(NWIN - 1, 1)
            writeout(NWIN - 1, 1)

    return k(a, b, idxr, idxc)


def _sc_gather_pos(p, idxr, idxc):
    @functools.partial(
        pl.kernel,
        out_type=(
            jax.ShapeDtypeStruct((E_HALF // 8, 8 * 16), f32),
            jax.ShapeDtypeStruct((E_HALF // 8, 8 * 16), f32),
        ),
        mesh=_mesh,
        compiler_params=pltpu.CompilerParams(use_tc_tiling_on_sc=False),
        scratch_types=[
            pltpu.VMEM((NWIN, W), jnp.int32),
            pltpu.VMEM((NWIN, W), jnp.int32),
            pltpu.VMEM((2, W, 16), f32),
            pltpu.VMEM((2, W, 16), f32),
            pltpu.VMEM((W // 8, 8 * 16), f32),
            pltpu.VMEM((W // 8, 8 * 16), f32),
            pltpu.SemaphoreType.DMA,
            pltpu.SemaphoreType.DMA,
        ],
    )
    def k(p_h, ir_h, ic_h, pr_h, pc_h,
          ir_v, ic_v, ra, rb, ta, tb, sem0, sem1):
        c = lax.axis_index("c")
        s = lax.axis_index("s")
        wid = s * NC + c
        pltpu.sync_copy(ir_h.at[wid], ir_v)
        pltpu.sync_copy(ic_h.at[wid], ic_v)
        base = wid * EPT
        sems = (sem0, sem1)

        def fire(w, sl):
            pltpu.async_copy(p_h.at[ir_v.at[w]], ra.at[sl], sems[sl])
            pltpu.async_copy(p_h.at[ic_v.at[w]], rb.at[sl], sems[sl])

        def drain(w, sl):
            pltpu.make_async_copy(p_h.at[ir_v.at[w]], ra.at[sl], sems[sl]).wait()
            pltpu.make_async_copy(p_h.at[ic_v.at[w]], rb.at[sl], sems[sl]).wait()

        def writeout(w, sl):
            def pack(q, carry):
                for ci in range(8):
                    ta[q, pl.ds(16 * ci, 16)] = ra[sl, 8 * q + ci, :]
                    tb[q, pl.ds(16 * ci, 16)] = rb[sl, 8 * q + ci, :]
                return carry

            lax.fori_loop(0, W // 8, pack, 0)
            off8 = (base + w * W) // 8
            pltpu.sync_copy(ta, pr_h.at[pl.ds(off8, W // 8)])
            pltpu.sync_copy(tb, pc_h.at[pl.ds(off8, W // 8)])

        fire(0, 0)

        def body(i, carry):
            w1 = 2 * i + 1
            fire(w1, 1)
            drain(w1 - 1, 0)
            writeout(w1 - 1, 0)
            w2 = 2 * i + 2
            fire(w2, 0)
            drain(w2 - 1, 1)
            writeout(w2 - 1, 1)
            return carry

        if NWIN % 2:
            lax.fori_loop(0, (NWIN - 1) // 2, body, 0)
            drain(NWIN - 1, 0)
            writeout(NWIN - 1, 0)
        else:
            lax.fori_loop(0, (NWIN - 2) // 2, body, 0)
            fire(NWIN - 1, 1)
            drain(NWIN - 2, 0)
            writeout(NWIN - 2, 0)
            drain(NWIN - 1, 1)
            writeout(NWIN - 1, 1)

    return k(p, idxr, idxc)


def _sc_scatter(u, idxs, tc_tiling):
    d = u.shape[1]

    def body(u_h, ix_h, ou_h, ix_v, ub, acc_u, sem0, sem1):
        c = lax.axis_index("c")
        s = lax.axis_index("s")
        wid = s * NC + c
        r0 = s * RPS

        def zrow(i, carry):
            for j in range(d // 16):
                ub[0, i, pl.ds(j * 16, 16)] = jnp.zeros((16,), f32)
            return carry

        lax.fori_loop(0, W, zrow, 0)
        for k in range(RPS // W):
            pltpu.sync_copy(ub.at[0], acc_u.at[pl.ds(r0 + k * W, W)])
        pltpu.sync_copy(ix_h.at[wid], ix_v)
        plsc.subcore_barrier()
        sems = (sem0, sem1)

        def fire(w, sl):
            off = wid * EPT + w * W
            pltpu.async_copy(u_h.at[pl.ds(off, W)], ub.at[sl], sems[sl])

        def drain(w, sl):
            off = wid * EPT + w * W
            pltpu.make_async_copy(u_h.at[pl.ds(off, W)], ub.at[sl],
                                  sems[sl]).wait()

        def consume(w, sl):
            pltpu.sync_copy(ub.at[sl], acc_u.at[ix_v.at[w]], add=True)

        fire(0, 0)

        def w_body(i, carry):
            w1 = 2 * i + 1
            fire(w1, 1)
            drain(w1 - 1, 0)
            consume(w1 - 1, 0)
            w2 = 2 * i + 2
            fire(w2, 0)
            drain(w2 - 1, 1)
            consume(w2 - 1, 1)
            return carry

        if NWIN % 2:
            lax.fori_loop(0, (NWIN - 1) // 2, w_body, 0)
            drain(NWIN - 1, 0)
            consume(NWIN - 1, 0)
        else:
            lax.fori_loop(0, (NWIN - 2) // 2, w_body, 0)
            fire(NWIN - 1, 1)
            drain(NWIN - 2, 0)
            consume(NWIN - 2, 0)
            drain(NWIN - 1, 1)
            consume(NWIN - 1, 1)
        plsc.subcore_barrier()
        pltpu.sync_copy(acc_u.at[pl.ds(r0, RPS)], ou_h.at[c, pl.ds(r0, RPS)])

    kfn = functools.partial(
        pl.kernel,
        out_type=jax.ShapeDtypeStruct((NC, NACC, d), f32),
        mesh=_mesh,
        compiler_params=pltpu.CompilerParams(use_tc_tiling_on_sc=tc_tiling),
        scratch_types=[
            pltpu.VMEM((NWIN, W), jnp.int32),
            pltpu.VMEM((2, W, d), f32),
            pltpu.VMEM_SHARED((NACC, d), f32),
            pltpu.SemaphoreType.DMA,
            pltpu.SemaphoreType.DMA,
        ])(body)
    return kfn(u, idxs)


def _full(shape):
    return pl.BlockSpec(shape, lambda i: tuple(0 for _ in shape))


def _prep(x, wemb, bemb, w1a, w1b):
    def body(x_b, we, be, wa, wb, h_o, a_o, b_o):
        h = jnp.dot(x_b[...], we[...], preferred_element_type=f32) + be[...]
        h_o[...] = h
        a_o[...] = jnp.dot(h, wa[...], preferred_element_type=f32)
        b_o[...] = jnp.dot(h, wb[...], preferred_element_type=f32)

    n_spec = pl.BlockSpec((BN, HID), lambda i: (i, 0))
    return pl.pallas_call(
        body,
        grid=(N_NODES // BN,),
        in_specs=[n_spec, _full((HID, HID)), _full((1, HID)),
                  _full((HID, HID)), _full((HID, HID))],
        out_specs=[n_spec] * 3,
        out_shape=[jax.ShapeDtypeStruct((N_NODES, HID), f32)] * 3,
    )(x, wemb, bemb, w1a, w1b)


def _edge(g, prp, pcp, eap, masks, w1e_stack, b1, wr, w2, b2, coord):
    def body(*refs):
        if coord is not None:
            (g_b, prp_b, pcp_b, eap_b, sel_r, gm_r, pm_r, m8_r,
             w1es_, b1_, wr_, w2_, b2_, wc1_, bc1_, wc2_, u_o, t_o) = refs
        else:
            (g_b, prp_b, pcp_b, eap_b, sel_r, gm_r, pm_r, m8_r,
             w1es_, b1_, wr_, w2_, b2_, u_o) = refs
        bf16 = jnp.bfloat16
        sel = sel_r[...].astype(bf16)
        gm = gm_r[...]

        dp = prp_b[...] - pcp_b[...]
        rdd = jnp.dot(sel, (dp * dp).astype(bf16),
                      preferred_element_type=f32) * gm
        radial = jnp.sum(rdd, axis=1, keepdims=True)
        rea = jnp.dot(sel, eap_b[...].astype(bf16),
                      preferred_element_type=f32) * gm
        ea_term = jnp.dot(rea.astype(bf16), w1es_[...].astype(bf16),
                          preferred_element_type=f32)
        pre = g_b[...] + radial * wr_[...] + ea_term + b1_[...]
        u = _silu(pre)
        m = _silu(jnp.dot(u.astype(bf16), w2_[...].astype(bf16),
                          preferred_element_type=f32) + b2_[...])
        u_o[...] = m
        if coord is not None:
            cc = _silu(jnp.dot(m.astype(bf16), wc1_[...].astype(bf16),
                               preferred_element_type=f32) + bc1_[...])
            sclr = jnp.sum(cc * wc2_[...], axis=1, keepdims=True)
            sg = lax.dot_general(sel, sclr * pm_r[...],
                                 (((0,), (0,)), ((), ())),
                                 preferred_element_type=f32)
            s16 = jnp.dot(sg, m8_r[...], preferred_element_type=f32)
            lane16 = lax.broadcasted_iota(jnp.int32, (BE // 8, HID), 1)
            t_o[...] = jnp.where(lane16 % 16 == 3, 1.0, dp * s16)

    e_spec = pl.BlockSpec((BE, HID), lambda i: (i, 0))
    p_spec = pl.BlockSpec((BE // 8, HID), lambda i: (i, 0))
    in_specs = [e_spec, p_spec, p_spec, p_spec,
                _full((BE, BE // 8)), _full((BE, HID)), _full((BE, 8)),
                _full((8, HID)),
                _full((HID, HID)), _full((1, HID)), _full((1, HID)),
                _full((HID, HID)), _full((1, HID))]
    args = [g, prp, pcp, eap] + list(masks) + [w1e_stack, b1, wr, w2, b2]
    out_specs = [e_spec]
    out_shape = [jax.ShapeDtypeStruct((E_HALF, HID), f32)]
    if coord is not None:
        in_specs += [_full((HID, HID)), _full((1, HID)), _full((1, HID))]
        args += list(coord)
        out_specs.append(p_spec)
        out_shape.append(jax.ShapeDtypeStruct((E_HALF // 8, HID), f32))
    res = pl.pallas_call(
        body, grid=(E_HALF // BE,), in_specs=in_specs,
        out_specs=out_specs, out_shape=out_shape,
    )(*args)
    return res if coord is not None else res[0]


def _node1(pu, pt, h, posp, wna, wnb, bn1, wn2, bn2, wa2, wb2):
    def body(pu_b, pt_b, h_b, pp_b, wna_, wnb_, bn1_, wn2_, bn2_,
             wa2_, wb2_, h2_o, a2_o, b2_o, p2_o):
        agg = pu_b[0] + pu_b[1] + pu_b[2] + pu_b[3]
        t = pt_b[0] + pt_b[1] + pt_b[2] + pt_b[3]
        cnt = jnp.maximum(t[:, 3:4], 1.0)
        lane = lax.broadcasted_iota(jnp.int32, (BN, 16), 1)
        p2_o[...] = pp_b[...] + jnp.where(lane < 3, t / cnt, 0.0)
        pre = (jnp.dot(h_b[...], wna_[...], preferred_element_type=f32)
               + jnp.dot(agg, wnb_[...], preferred_element_type=f32)
               + bn1_[...])
        hn = (jnp.dot(_silu(pre), wn2_[...], preferred_element_type=f32)
              + bn2_[...])
        h2 = h_b[...] + hn
        h2_o[...] = h2
        a2_o[...] = jnp.dot(h2, wa2_[...], preferred_element_type=f32)
        b2_o[...] = jnp.dot(h2, wb2_[...], preferred_element_type=f32)

    n_spec = pl.BlockSpec((BN, HID), lambda i: (i, 0))
    s_spec = pl.BlockSpec((BN, 16), lambda i: (i, 0))
    pu_spec = pl.BlockSpec((NH * NC, BN, HID), lambda i: (0, i, 0))
    pt_spec = pl.BlockSpec((NH * NC, BN, 16), lambda i: (0, i, 0))
    return pl.pallas_call(
        body,
        grid=(N_NODES // BN,),
        in_specs=[pu_spec, pt_spec, n_spec, s_spec,
                  _full((HID, HID)), _full((HID, HID)), _full((1, HID)),
                  _full((HID, HID)), _full((1, HID)),
                  _full((HID, HID)), _full((HID, HID))],
        out_specs=[n_spec, n_spec, n_spec, s_spec],
        out_shape=[jax.ShapeDtypeStruct((N_NODES, HID), f32)] * 3
        + [jax.ShapeDtypeStruct((N_NODES, 16), f32)],
    )(pu, pt, h, posp, wna, wnb, bn1, wn2, bn2, wa2, wb2)


def _node2(pu, h, wna, wnb, bn1, wn2, bn2, wo, bo):
    def body(pu_b, h_b, wna_, wnb_, bn1_, wn2_, bn2_, wo_, bo_, out_o):
        agg = pu_b[0] + pu_b[1] + pu_b[2] + pu_b[3]
        pre = (jnp.dot(h_b[...], wna_[...], preferred_element_type=f32)
               + jnp.dot(agg, wnb_[...], preferred_element_type=f32)
               + bn1_[...])
        hn = (jnp.dot(_silu(pre), wn2_[...], preferred_element_type=f32)
              + bn2_[...])
        h2 = h_b[...] + hn
        out_o[...] = jnp.dot(h2, wo_[...], preferred_element_type=f32) + bo_[...]

    n_spec = pl.BlockSpec((BN, HID), lambda i: (i, 0))
    pu_spec = pl.BlockSpec((NH * NC, BN, HID), lambda i: (0, i, 0))
    return pl.pallas_call(
        body,
        grid=(N_NODES // BN,),
        in_specs=[pu_spec, n_spec,
                  _full((HID, HID)), _full((HID, HID)), _full((1, HID)),
                  _full((HID, HID)), _full((1, HID)),
                  _full((HID, HID)), _full((1, HID))],
        out_specs=[n_spec],
        out_shape=[jax.ShapeDtypeStruct((N_NODES, HID), f32)],
    )(pu, h, wna, wnb, bn1, wn2, bn2, wo, bo)[0]


def _row(v):
    return v.reshape(1, -1)


def kernel(x, pos, edge_attr, params, edge_index):
    row, col = edge_index[0], edge_index[1]
    e = row.shape[0]
    npad = E_PAD - e
    padg = (jnp.arange(npad, dtype=jnp.int32) * 97) % N_NODES
    pads = N_NODES + jnp.arange(npad, dtype=jnp.int32) % (NACC - N_NODES)
    idx_shape = (NH, NW, NWIN, W)
    rowg = jnp.concatenate([row, padg]).reshape(idx_shape)
    colg = jnp.concatenate([col, padg]).reshape(idx_shape)
    rows = jnp.concatenate([row, pads]).reshape(idx_shape)
    eap = jnp.concatenate(
        [jnp.reshape(edge_attr, (e // 8, 8 * ED)),
         jnp.zeros((npad // 8, 8 * ED), f32)],
        axis=0).reshape(NH, E_HALF // 8, 8 * ED)
    posp = jnp.concatenate([pos, jnp.zeros((N_NODES, 13), f32)], axis=1)

    e_i = lax.broadcasted_iota(jnp.int32, (BE, BE // 8), 0)
    r_i = lax.broadcasted_iota(jnp.int32, (BE, BE // 8), 1)
    sel = jnp.where(e_i // 8 == r_i, 1.0, 0.0).astype(f32)
    e_j = lax.broadcasted_iota(jnp.int32, (BE, HID), 0)
    c_j = lax.broadcasted_iota(jnp.int32, (BE, HID), 1)
    gm = jnp.where(c_j // 16 == e_j % 8, 1.0, 0.0).astype(f32)
    g_i = lax.broadcasted_iota(jnp.int32, (BE, 8), 1)
    e_k = lax.broadcasted_iota(jnp.int32, (BE, 8), 0)
    pm = jnp.where(e_k % 8 == g_i, 1.0, 0.0).astype(f32)
    c_m = lax.broadcasted_iota(jnp.int32, (8, HID), 1)
    g_m = lax.broadcasted_iota(jnp.int32, (8, HID), 0)
    m8 = jnp.where(c_m // 16 == g_m, 1.0, 0.0).astype(f32)
    masks = (sel, gm, pm, m8)

    lw = []
    for lp in params["layers"]:
        w1 = lp["edge_mlp"][0]["W"]
        lw.append(dict(
            wa=w1[:HID], wb=w1[HID:2 * HID], wr=_row(w1[2 * HID]),
            w1es=jnp.concatenate([w1[2 * HID + 1:]] * 8, axis=0),
            b1=_row(lp["edge_mlp"][0]["b"]),
            w2=lp["edge_mlp"][1]["W"], b2=_row(lp["edge_mlp"][1]["b"]),
            wc1=lp["coord_mlp"][0]["W"], bc1=_row(lp["coord_mlp"][0]["b"]),
            wc2=_row(lp["coord_mlp"][1]["W"][:, 0]),
            wna=lp["node_mlp"][0]["W"][:HID],
            wnb=lp["node_mlp"][0]["W"][HID:],
            bn1=_row(lp["node_mlp"][0]["b"]),
            wn2=lp["node_mlp"][1]["W"], bn2=_row(lp["node_mlp"][1]["b"]),
        ))

    h, a1, b1t = _prep(x, params["emb_in"]["W"], _row(params["emb_in"]["b"]),
                       lw[0]["wa"], lw[0]["wb"])

    pu1, pt1 = [], []
    for hh in range(NH):
        gg = _sc_gather2(a1, b1t, rowg[hh], colg[hh], HID, True)
        prg, pcg = _sc_gather_pos(posp, rowg[hh], colg[hh])
        u1, t1 = _edge(gg, prg, pcg,
                       eap[hh], masks, lw[0]["w1es"],
                       lw[0]["b1"], lw[0]["wr"], lw[0]["w2"], lw[0]["b2"],
                       (lw[0]["wc1"], lw[0]["bc1"], lw[0]["wc2"]))
        pu1.append(_sc_scatter(u1, rows[hh], True))
        pt1.append(_sc_scatter(jnp.reshape(t1, (E_HALF, 16)), rows[hh],
                               False))
    h2, a2, b2t, posp2 = _node1(jnp.concatenate(pu1), jnp.concatenate(pt1),
                                h, posp,
                                lw[0]["wna"], lw[0]["wnb"], lw[0]["bn1"],
                                lw[0]["wn2"], lw[0]["bn2"],
                                lw[1]["wa"], lw[1]["wb"])

    pu2 = []
    for hh in range(NH):
        gg2 = _sc_gather2(a2, b2t, rowg[hh], colg[hh], HID, True)
        prg2, pcg2 = _sc_gather_pos(posp2, rowg[hh], colg[hh])
        u2 = _edge(gg2, prg2, pcg2,
                   eap[hh], masks, lw[1]["w1es"], lw[1]["b1"],
                   lw[1]["wr"], lw[1]["w2"], lw[1]["b2"], None)
        pu2.append(_sc_scatter(u2, rows[hh], True))
    out = _node2(jnp.concatenate(pu2), h2,
                 lw[1]["wna"], lw[1]["wnb"], lw[1]["bn1"],
                 lw[1]["wn2"], lw[1]["bn2"],
                 params["emb_out"]["W"], _row(params["emb_out"]["b"]))
    return out


# speedup vs baseline: 1.1173x; 1.0001x over previous
import functools

import jax
import jax.numpy as jnp
from jax import lax
from jax.experimental import pallas as pl
from jax.experimental.pallas import tpu as pltpu
from jax.experimental.pallas import tpu_sc as plsc

N_NODES = 10000
HID = 128
ED = 16
NC, NS = 2, 16
NW = NC * NS
W = 128
NH = 2
NWIN = 40
EPT = W * NWIN
E_HALF = NW * EPT
E_PAD = NH * E_HALF
NACC = 10240
RPS = NACC // NS
BE = 1024
BN = 1000
f32 = jnp.float32

_mesh = plsc.VectorSubcoreMesh(core_axis_name="c", subcore_axis_name="s")


def _silu(x):
    return x * jax.nn.sigmoid(x)


def _sc_gather2(a, b, idxr, idxc, d, tc_tiling):
    @functools.partial(
        pl.kernel,
        out_type=jax.ShapeDtypeStruct((E_HALF, d), f32),
        mesh=_mesh,
        compiler_params=pltpu.CompilerParams(use_tc_tiling_on_sc=tc_tiling),
        scratch_types=[
            pltpu.VMEM((NWIN, W), jnp.int32),
            pltpu.VMEM((NWIN, W), jnp.int32),
            pltpu.VMEM((2, W, d), f32),
            pltpu.VMEM((2, W, d), f32),
            pltpu.SemaphoreType.DMA,
            pltpu.SemaphoreType.DMA,
        ],
    )
    def k(a_h, b_h, ir_h, ic_h, ga_h,
          ir_v, ic_v, ra, rb, sem0, sem1):
        c = lax.axis_index("c")
        s = lax.axis_index("s")
        wid = s * NC + c
        pltpu.sync_copy(ir_h.at[wid], ir_v)
        pltpu.sync_copy(ic_h.at[wid], ic_v)
        base = wid * EPT
        sems = (sem0, sem1)

        def fire(w, sl):
            pltpu.async_copy(a_h.at[ir_v.at[w]], ra.at[sl], sems[sl])
            pltpu.async_copy(b_h.at[ic_v.at[w]], rb.at[sl], sems[sl])

        def drain(w, sl):
            pltpu.make_async_copy(a_h.at[ir_v.at[w]], ra.at[sl], sems[sl]).wait()
            pltpu.make_async_copy(b_h.at[ic_v.at[w]], rb.at[sl], sems[sl]).wait()

        def writeout(w, sl):
            def addrow(i, carry):
                for j in range(d // 16):
                    ra[sl, i, pl.ds(16 * j, 16)] = (
                        ra[sl, i, pl.ds(16 * j, 16)]
                        + rb[sl, i, pl.ds(16 * j, 16)])
                return carry

            lax.fori_loop(0, W, addrow, 0)
            off = base + w * W
            pltpu.sync_copy(ra.at[sl], ga_h.at[pl.ds(off, W)])

        fire(0, 0)

        def body(i, carry):
            w1 = 2 * i + 1
            fire(w1, 1)
            drain(w1 - 1, 0)
            writeout(w1 - 1, 0)
            w2 = 2 * i + 2
            fire(w2, 0)
            drain(w2 - 1, 1)
            writeout(w2 - 1, 1)
            return carry

        if NWIN % 2:
            lax.fori_loop(0, (NWIN - 1) // 2, body, 0)
            drain(NWIN - 1, 0)
            writeout(NWIN - 1, 0)
        else:
            lax.fori_loop(0, (NWIN - 2) // 2, body, 0)
            fire(NWIN - 1, 1)
            drain(NWIN - 2, 0)
            writeout(NWIN - 2, 0)
            drain(NWIN - 1, 1)
            writeout(NWIN - 1, 1)

    return k(a, b, idxr, idxc)


def _sc_gather_pos(p, idxr, idxc):
    @functools.partial(
        pl.kernel,
        out_type=(
            jax.ShapeDtypeStruct((E_HALF // 8, 8 * 16), f32),
            jax.ShapeDtypeStruct((E_HALF // 8, 8 * 16), f32),
        ),
        mesh=_mesh,
        compiler_params=pltpu.CompilerParams(use_tc_tiling_on_sc=False),
        scratch_types=[
            pltpu.VMEM((NWIN, W), jnp.int32),
            pltpu.VMEM((NWIN, W), jnp.int32),
            pltpu.VMEM((2, W, 16), f32),
            pltpu.VMEM((2, W, 16), f32),
            pltpu.VMEM((W // 8, 8 * 16), f32),
            pltpu.VMEM((W // 8, 8 * 16), f32),
            pltpu.SemaphoreType.DMA,
            pltpu.SemaphoreType.DMA,
        ],
    )
    def k(p_h, ir_h, ic_h, pr_h, pc_h,
          ir_v, ic_v, ra, rb, ta, tb, sem0, sem1):
        c = lax.axis_index("c")
        s = lax.axis_index("s")
        wid = s * NC + c
        pltpu.sync_copy(ir_h.at[wid], ir_v)
        pltpu.sync_copy(ic_h.at[wid], ic_v)
        base = wid * EPT
        sems = (sem0, sem1)

        def fire(w, sl):
            pltpu.async_copy(p_h.at[ir_v.at[w]], ra.at[sl], sems[sl])
            pltpu.async_copy(p_h.at[ic_v.at[w]], rb.at[sl], sems[sl])

        def drain(w, sl):
            pltpu.make_async_copy(p_h.at[ir_v.at[w]], ra.at[sl], sems[sl]).wait()
            pltpu.make_async_copy(p_h.at[ic_v.at[w]], rb.at[sl], sems[sl]).wait()

        def writeout(w, sl):
            def pack(q, carry):
                for ci in range(8):
                    ta[q, pl.ds(16 * ci, 16)] = ra[sl, 8 * q + ci, :]
                    tb[q, pl.ds(16 * ci, 16)] = rb[sl, 8 * q + ci, :]
                return carry

            lax.fori_loop(0, W // 8, pack, 0)
            off8 = (base + w * W) // 8
            pltpu.sync_copy(ta, pr_h.at[pl.ds(off8, W // 8)])
            pltpu.sync_copy(tb, pc_h.at[pl.ds(off8, W // 8)])

        fire(0, 0)

        def body(i, carry):
            w1 = 2 * i + 1
            fire(w1, 1)
            drain(w1 - 1, 0)
            writeout(w1 - 1, 0)
            w2 = 2 * i + 2
            fire(w2, 0)
            drain(w2 - 1, 1)
            writeout(w2 - 1, 1)
            return carry

        if NWIN % 2:
            lax.fori_loop(0, (NWIN - 1) // 2, body, 0)
            drain(NWIN - 1, 0)
            writeout(NWIN - 1, 0)
        else:
            lax.fori_loop(0, (NWIN - 2) // 2, body, 0)
            fire(NWIN - 1, 1)
            drain(NWIN - 2, 0)
            writeout(NWIN - 2, 0)
            drain(NWIN - 1, 1)
            writeout(NWIN - 1, 1)

    return k(p, idxr, idxc)


def _sc_scatter(u, idxs, tc_tiling):
    d = u.shape[1]

    def body(u_h, ix_h, ou_h, ix_v, ub, acc_u, sem0, sem1):
        c = lax.axis_index("c")
        s = lax.axis_index("s")
        wid = s * NC + c
        r0 = s * RPS

        def zrow(i, carry):
            for j in range(d // 16):
                ub[0, i, pl.ds(j * 16, 16)] = jnp.zeros((16,), f32)
            return carry

        lax.fori_loop(0, W, zrow, 0)
        for k in range(RPS // W):
            pltpu.sync_copy(ub.at[0], acc_u.at[pl.ds(r0 + k * W, W)])
        pltpu.sync_copy(ix_h.at[wid], ix_v)
        plsc.subcore_barrier()
        sems = (sem0, sem1)

        def fire(w, sl):
            off = wid * EPT + w * W
            pltpu.async_copy(u_h.at[pl.ds(off, W)], ub.at[sl], sems[sl])

        def drain(w, sl):
            off = wid * EPT + w * W
            pltpu.make_async_copy(u_h.at[pl.ds(off, W)], ub.at[sl],
                                  sems[sl]).wait()

        def consume(w, sl):
            pltpu.sync_copy(ub.at[sl], acc_u.at[ix_v.at[w]], add=True)

        fire(0, 0)

        def w_body(i, carry):
            w1 = 2 * i + 1
            fire(w1, 1)
            drain(w1 - 1, 0)
            consume(w1 - 1, 0)
            w2 = 2 * i + 2
            fire(w2, 0)
            drain(w2 - 1, 1)
            consume(w2 - 1, 1)
            return carry

        if NWIN % 2:
            lax.fori_loop(0, (NWIN - 1) // 2, w_body, 0)
            drain(NWIN - 1, 0)
            consume(NWIN - 1, 0)
        else:
            lax.fori_loop(0, (NWIN - 2) // 2, w_body, 0)
            fire(NWIN - 1, 1)
            drain(NWIN - 2, 0)
            consume(NWIN - 2, 0)
            drain(NWIN - 1, 1)
            consume(NWIN - 1, 1)
        plsc.subcore_barrier()
        pltpu.sync_copy(acc_u.at[pl.ds(r0, RPS)], ou_h.at[c, pl.ds(r0, RPS)])

    kfn = functools.partial(
        pl.kernel,
        out_type=jax.ShapeDtypeStruct((NC, NACC, d), f32),
        mesh=_mesh,
        compiler_params=pltpu.CompilerParams(use_tc_tiling_on_sc=tc_tiling),
        scratch_types=[
            pltpu.VMEM((NWIN, W), jnp.int32),
            pltpu.VMEM((2, W, d), f32),
            pltpu.VMEM_SHARED((NACC, d), f32),
            pltpu.SemaphoreType.DMA,
            pltpu.SemaphoreType.DMA,
        ])(body)
    return kfn(u, idxs)


def _full(shape):
    return pl.BlockSpec(shape, lambda i: tuple(0 for _ in shape))


def _prep(x, wemb, bemb, w1a, w1b):
    def body(x_b, we, be, wa, wb, h_o, a_o, b_o):
        h = jnp.dot(x_b[...], we[...], preferred_element_type=f32) + be[...]
        h_o[...] = h
        a_o[...] = jnp.dot(h, wa[...], preferred_element_type=f32)
        b_o[...] = jnp.dot(h, wb[...], preferred_element_type=f32)

    n_spec = pl.BlockSpec((BN, HID), lambda i: (i, 0))
    return pl.pallas_call(
        body,
        grid=(N_NODES // BN,),
        in_specs=[n_spec, _full((HID, HID)), _full((1, HID)),
                  _full((HID, HID)), _full((HID, HID))],
        out_specs=[n_spec] * 3,
        out_shape=[jax.ShapeDtypeStruct((N_NODES, HID), f32)] * 3,
    )(x, wemb, bemb, w1a, w1b)


def _edge(g, prp, pcp, eap, masks, w1e_stack, b1, wr, w2, b2, coord):
    def body(*refs):
        if coord is not None:
            (g_b, prp_b, pcp_b, eap_b, sel_r, gm_r, pm_r, m8_r,
             w1es_, b1_, wr_, w2_, b2_, wc1_, bc1_, wc2_, u_o, t_o) = refs
        else:
            (g_b, prp_b, pcp_b, eap_b, sel_r, gm_r, pm_r, m8_r,
             w1es_, b1_, wr_, w2_, b2_, u_o) = refs
        bf16 = jnp.bfloat16
        sel = sel_r[...].astype(bf16)
        gm = gm_r[...]

        dp = prp_b[...] - pcp_b[...]
        rdd = jnp.dot(sel, (dp * dp).astype(bf16),
                      preferred_element_type=f32) * gm
        radial = jnp.sum(rdd, axis=1, keepdims=True)
        rea = jnp.dot(sel, eap_b[...].astype(bf16),
                      preferred_element_type=f32) * gm
        ea_term = jnp.dot(rea.astype(bf16), w1es_[...].astype(bf16),
                          preferred_element_type=f32)
        pre = g_b[...] + radial * wr_[...] + ea_term + b1_[...]
        u = _silu(pre)
        m = _silu(jnp.dot(u.astype(bf16), w2_[...].astype(bf16),
                          preferred_element_type=f32) + b2_[...])
        u_o[...] = m
        if coord is not None:
            cc = _silu(jnp.dot(m.astype(bf16), wc1_[...].astype(bf16),
                               preferred_element_type=f32) + bc1_[...])
            sclr = jnp.sum(cc * wc2_[...], axis=1, keepdims=True)
            sg = lax.dot_general(sel, sclr * pm_r[...],
                                 (((0,), (0,)), ((), ())),
                                 preferred_element_type=f32)
            s16 = jnp.dot(sg, m8_r[...], preferred_element_type=f32)
            lane16 = lax.broadcasted_iota(jnp.int32, (BE // 8, HID), 1)
            t_o[...] = jnp.where(lane16 % 16 == 3, 1.0, dp * s16)

    e_spec = pl.BlockSpec((BE, HID), lambda i: (i, 0))
    p_spec = pl.BlockSpec((BE // 8, HID), lambda i: (i, 0))
    in_specs = [e_spec, p_spec, p_spec, p_spec,
                _full((BE, BE // 8)), _full((BE, HID)), _full((BE, 8)),
                _full((8, HID)),
                _full((HID, HID)), _full((1, HID)), _full((1, HID)),
                _full((HID, HID)), _full((1, HID))]
    args = [g, prp, pcp, eap] + list(masks) + [w1e_stack, b1, wr, w2, b2]
    out_specs = [e_spec]
    out_shape = [jax.ShapeDtypeStruct((E_HALF, HID), f32)]
    if coord is not None:
        in_specs += [_full((HID, HID)), _full((1, HID)), _full((1, HID))]
        args += list(coord)
        out_specs.append(p_spec)
        out_shape.append(jax.ShapeDtypeStruct((E_HALF // 8, HID), f32))
    res = pl.pallas_call(
        body, grid=(E_HALF // BE,), in_specs=in_specs,
        out_specs=out_specs, out_shape=out_shape,
    )(*args)
    return res if coord is not None else res[0]


def _node1(pu, pt, h, posp, wna, wnb, bn1, wn2, bn2, wa2, wb2):
    def body(pu_b, pt_b, h_b, pp_b, wna_, wnb_, bn1_, wn2_, bn2_,
             wa2_, wb2_, h2_o, a2_o, b2_o, p2_o):
        agg = pu_b[0] + pu_b[1] + pu_b[2] + pu_b[3]
        t = pt_b[0] + pt_b[1] + pt_b[2] + pt_b[3]
        cnt = jnp.maximum(t[:, 3:4], 1.0)
        lane = lax.broadcasted_iota(jnp.int32, (BN, 16), 1)
        p2_o[...] = pp_b[...] + jnp.where(lane < 3, t / cnt, 0.0)
        pre = (jnp.dot(h_b[...], wna_[...], preferred_element_type=f32)
               + jnp.dot(agg, wnb_[...], preferred_element_type=f32)
               + bn1_[...])
        hn = (jnp.dot(_silu(pre), wn2_[...], preferred_element_type=f32)
              + bn2_[...])
        h2 = h_b[...] + hn
        h2_o[...] = h2
        a2_o[...] = jnp.dot(h2, wa2_[...], preferred_element_type=f32)
        b2_o[...] = jnp.dot(h2, wb2_[...], preferred_element_type=f32)

    n_spec = pl.BlockSpec((BN, HID), lambda i: (i, 0))
    s_spec = pl.BlockSpec((BN, 16), lambda i: (i, 0))
    pu_spec = pl.BlockSpec((NH * NC, BN, HID), lambda i: (0, i, 0))
    pt_spec = pl.BlockSpec((NH * NC, BN, 16), lambda i: (0, i, 0))
    return pl.pallas_call(
        body,
        grid=(N_NODES // BN,),
        in_specs=[pu_spec, pt_spec, n_spec, s_spec,
                  _full((HID, HID)), _full((HID, HID)), _full((1, HID)),
                  _full((HID, HID)), _full((1, HID)),
                  _full((HID, HID)), _full((HID, HID))],
        out_specs=[n_spec, n_spec, n_spec, s_spec],
        out_shape=[jax.ShapeDtypeStruct((N_NODES, HID), f32)] * 3
        + [jax.ShapeDtypeStruct((N_NODES, 16), f32)],
    )(pu, pt, h, posp, wna, wnb, bn1, wn2, bn2, wa2, wb2)


def _node2(pu, h, wna, wnb, bn1, wn2, bn2, wo, bo):
    def body(pu_b, h_b, wna_, wnb_, bn1_, wn2_, bn2_, wo_, bo_, out_o):
        agg = pu_b[0] + pu_b[1] + pu_b[2] + pu_b[3]
        pre = (jnp.dot(h_b[...], wna_[...], preferred_element_type=f32)
               + jnp.dot(agg, wnb_[...], preferred_element_type=f32)
               + bn1_[...])
        hn = (jnp.dot(_silu(pre), wn2_[...], preferred_element_type=f32)
              + bn2_[...])
        h2 = h_b[...] + hn
        out_o[...] = jnp.dot(h2, wo_[...], preferred_element_type=f32) + bo_[...]

    n_spec = pl.BlockSpec((BN, HID), lambda i: (i, 0))
    pu_spec = pl.BlockSpec((NH * NC, BN, HID), lambda i: (0, i, 0))
    return pl.pallas_call(
        body,
        grid=(N_NODES // BN,),
        in_specs=[pu_spec, n_spec,
                  _full((HID, HID)), _full((HID, HID)), _full((1, HID)),
                  _full((HID, HID)), _full((1, HID)),
                  _full((HID, HID)), _full((1, HID))],
        out_specs=[n_spec],
        out_shape=[jax.ShapeDtypeStruct((N_NODES, HID), f32)],
    )(pu, h, wna, wnb, bn1, wn2, bn2, wo, bo)[0]


def _row(v):
    return v.reshape(1, -1)


def kernel(x, pos, edge_attr, params, edge_index):
    row, col = edge_index[0], edge_index[1]
    e = row.shape[0]
    npad = E_PAD - e
    padg = (jnp.arange(npad, dtype=jnp.int32) * 97) % N_NODES
    pads = N_NODES + jnp.arange(npad, dtype=jnp.int32) % (NACC - N_NODES)
    idx_shape = (NH, NW, NWIN, W)
    rowg = jnp.concatenate([row, padg]).reshape(idx_shape)
    colg = jnp.concatenate([col, padg]).reshape(idx_shape)
    rows = jnp.concatenate([row, pads]).reshape(idx_shape)
    eap = jnp.concatenate(
        [jnp.reshape(edge_attr, (e // 8, 8 * ED)),
         jnp.zeros((npad // 8, 8 * ED), f32)],
        axis=0).reshape(NH, E_HALF // 8, 8 * ED)
    posp = jnp.concatenate([pos, jnp.zeros((N_NODES, 13), f32)], axis=1)

    e_i = lax.broadcasted_iota(jnp.int32, (BE, BE // 8), 0)
    r_i = lax.broadcasted_iota(jnp.int32, (BE, BE // 8), 1)
    sel = jnp.where(e_i // 8 == r_i, 1.0, 0.0).astype(f32)
    e_j = lax.broadcasted_iota(jnp.int32, (BE, HID), 0)
    c_j = lax.broadcasted_iota(jnp.int32, (BE, HID), 1)
    gm = jnp.where(c_j // 16 == e_j % 8, 1.0, 0.0).astype(f32)
    g_i = lax.broadcasted_iota(jnp.int32, (BE, 8), 1)
    e_k = lax.broadcasted_iota(jnp.int32, (BE, 8), 0)
    pm = jnp.where(e_k % 8 == g_i, 1.0, 0.0).astype(f32)
    c_m = lax.broadcasted_iota(jnp.int32, (8, HID), 1)
    g_m = lax.broadcasted_iota(jnp.int32, (8, HID), 0)
    m8 = jnp.where(c_m // 16 == g_m, 1.0, 0.0).astype(f32)
    masks = (sel, gm, pm, m8)

    lw = []
    for lp in params["layers"]:
        w1 = lp["edge_mlp"][0]["W"]
        lw.append(dict(
            wa=w1[:HID], wb=w1[HID:2 * HID], wr=_row(w1[2 * HID]),
            w1es=jnp.concatenate([w1[2 * HID + 1:]] * 8, axis=0),
            b1=_row(lp["edge_mlp"][0]["b"]),
            w2=lp["edge_mlp"][1]["W"], b2=_row(lp["edge_mlp"][1]["b"]),
            wc1=lp["coord_mlp"][0]["W"], bc1=_row(lp["coord_mlp"][0]["b"]),
            wc2=_row(lp["coord_mlp"][1]["W"][:, 0]),
            wna=lp["node_mlp"][0]["W"][:HID],
            wnb=lp["node_mlp"][0]["W"][HID:],
            bn1=_row(lp["node_mlp"][0]["b"]),
            wn2=lp["node_mlp"][1]["W"], bn2=_row(lp["node_mlp"][1]["b"]),
        ))

    h, a1, b1t = _prep(x, params["emb_in"]["W"], _row(params["emb_in"]["b"]),
                       lw[0]["wa"], lw[0]["wb"])

    pu1, pt1 = [], []
    for hh in range(NH):
        gg = _sc_gather2(a1, b1t, rowg[hh], colg[hh], HID, True)
        prg, pcg = _sc_gather_pos(posp, rowg[hh], colg[hh])
        u1, t1 = _edge(gg, prg, pcg,
                       eap[hh], masks, lw[0]["w1es"],
                       lw[0]["b1"], lw[0]["wr"], lw[0]["w2"], lw[0]["b2"],
                       (lw[0]["wc1"], lw[0]["bc1"], lw[0]["wc2"]))
        pu1.append(_sc_scatter(u1, rows[hh], True))
        pt1.append(_sc_scatter(jnp.reshape(t1, (E_HALF, 16)), rows[hh],
                               False))
    h2, a2, b2t, posp2 = _node1(jnp.concatenate(pu1), jnp.concatenate(pt1),
                                h, posp,
                                lw[0]["wna"], lw[0]["wnb"], lw[0]["bn1"],
                                lw[0]["wn2"], lw[0]["bn2"],
                                lw[1]["wa"], lw[1]["wb"])

    pu2 = []
    for hh in range(NH):
        gg2 = _sc_gather2(a2, b2t, rowg[hh], colg[hh], HID, True)
        prg2, pcg2 = _sc_gather_pos(posp2, rowg[hh], colg[hh])
        u2 = _edge(gg2, prg2, pcg2,
                   eap[hh], masks, lw[1]["w1es"], lw[1]["b1"],
                   lw[1]["wr"], lw[1]["w2"], lw[1]["b2"], None)
        pu2.append(_sc_scatter(u2, rows[hh], True))
    out = _node2(jnp.concatenate(pu2), h2,
                 lw[1]["wna"], lw[1]["wnb"], lw[1]["bn1"],
                 lw[1]["wn2"], lw[1]["bn2"],
                 params["emb_out"]["W"], _row(params["emb_out"]["b"]))
    return out
